# Initial kernel scaffold; baseline (speedup 1.0000x reference)
import jax
import jax.numpy as jnp
from jax.experimental import pallas as pl


def kernel(X, theta):
    raise NotImplementedError("write your pallas kernel here")


# speedup vs baseline: 3.0011x; 3.0011x over previous
import functools

import jax
import jax.numpy as jnp
from jax.experimental import pallas as pl
from jax.experimental.pallas import tpu as pltpu

RADIUS = 2.0
Y_ACC = 0.001
X_ACC = 0.001
NUM_DIMS = 8
TEMP = 50.0
EPS = 0.0001
BATCH = 2048
NY = int(2 * RADIUS / Y_ACC) + 1
NX = int(2 * RADIUS / X_ACC) + 1
NY_PAD = 4096
BLK_B = 64


def _fsm_kernel(x_ref, b_ref, y_ref, out_ref):
    x = x_ref[...]
    xp = jnp.clip(x, -RADIUS + EPS, RADIUS - EPS)
    idx = jnp.round((xp + RADIUS) / (2.0 * RADIUS) * (NX - 1))
    xg = -RADIUS + idx * (2.0 * RADIUS / (NX - 1))

    y = y_ref[...]
    bt = b_ref[...]

    d = xg[:, :, None] - y[None, :, :]
    s = jnp.exp(-d * d) - bt[None, :, :]

    mask = jax.lax.broadcasted_iota(jnp.int32, (1, 1, NY_PAD), 2) < NY
    s = jnp.where(mask, s, -1e30)

    m = jnp.max(s, axis=-1, keepdims=True)
    e = jnp.exp(TEMP * (s - m))
    num = jnp.sum(e * s, axis=-1)
    den = jnp.sum(e, axis=-1)
    out_ref[...] = jnp.sum(num / den, axis=-1, keepdims=True)


@jax.jit
def kernel(X, theta):
    y_grid = jnp.linspace(-RADIUS, RADIUS, NY, dtype=jnp.float32)
    y_pad = jnp.zeros((1, NY_PAD), jnp.float32).at[0, :NY].set(y_grid)
    b = jnp.concatenate(
        [jnp.zeros((1, NUM_DIMS), jnp.float32), theta], axis=0
    )
    bt = jnp.zeros((NUM_DIMS, NY_PAD), jnp.float32).at[:, :NY].set(b.T)

    grid = BATCH // BLK_B
    out = pl.pallas_call(
        _fsm_kernel,
        grid=(grid,),
        in_specs=[
            pl.BlockSpec((BLK_B, NUM_DIMS), lambda i: (i, 0)),
            pl.BlockSpec((NUM_DIMS, NY_PAD), lambda i: (0, 0)),
            pl.BlockSpec((1, NY_PAD), lambda i: (0, 0)),
        ],
        out_specs=pl.BlockSpec((BLK_B, 1), lambda i: (i, 0)),
        out_shape=jax.ShapeDtypeStruct((BATCH, 1), jnp.float32),
        compiler_params=pltpu.CompilerParams(
            dimension_semantics=("arbitrary",),
        ),
    )(X, bt, y_pad)
    return out.reshape(BATCH)


# speedup vs baseline: 3.0448x; 1.0146x over previous
import functools

import jax
import jax.numpy as jnp
from jax.experimental import pallas as pl
from jax.experimental.pallas import tpu as pltpu

RADIUS = 2.0
Y_ACC = 0.001
X_ACC = 0.001
NUM_DIMS = 8
TEMP = 50.0
EPS = 0.0001
BATCH = 2048
NY = int(2 * RADIUS / Y_ACC) + 1
NX = int(2 * RADIUS / X_ACC) + 1
NY_PAD = 4096
BLK_B = 64


def _fsm_kernel(x_ref, b_ref, y_ref, out_ref):
    x = x_ref[...]
    xp = jnp.clip(x, -RADIUS + EPS, RADIUS - EPS)
    idx = jnp.round((xp + RADIUS) / (2.0 * RADIUS) * (NX - 1))
    xg = -RADIUS + idx * (2.0 * RADIUS / (NX - 1))

    y = y_ref[...]
    bt = b_ref[...]

    d = xg[:, :, None] - y[None, :, :]
    s = jnp.exp(-d * d) - bt[None, :, :]

    mask = jax.lax.broadcasted_iota(jnp.int32, (1, 1, NY_PAD), 2) < NY
    s = jnp.where(mask, s, -1e30)

    m = jnp.max(s, axis=-1, keepdims=True)
    e = jnp.exp(TEMP * (s - m))
    num = jnp.sum(e * s, axis=-1)
    den = jnp.sum(e, axis=-1)
    out_ref[...] = jnp.sum(num / den, axis=-1, keepdims=True)


@jax.jit
def kernel(X, theta):
    y_grid = jnp.linspace(-RADIUS, RADIUS, NY, dtype=jnp.float32)
    y_pad = jnp.zeros((1, NY_PAD), jnp.float32).at[0, :NY].set(y_grid)
    b = jnp.concatenate(
        [jnp.zeros((1, NUM_DIMS), jnp.float32), theta], axis=0
    )
    bt = jnp.zeros((NUM_DIMS, NY_PAD), jnp.float32).at[:, :NY].set(b.T)

    grid = BATCH // BLK_B
    out = pl.pallas_call(
        _fsm_kernel,
        grid=(grid,),
        in_specs=[
            pl.BlockSpec((BLK_B, NUM_DIMS), lambda i: (i, 0)),
            pl.BlockSpec((NUM_DIMS, NY_PAD), lambda i: (0, 0)),
            pl.BlockSpec((1, NY_PAD), lambda i: (0, 0)),
        ],
        out_specs=pl.BlockSpec((BLK_B, 1), lambda i: (i, 0)),
        out_shape=jax.ShapeDtypeStruct((BATCH, 1), jnp.float32),
        compiler_params=pltpu.CompilerParams(
            dimension_semantics=("parallel",),
        ),
    )(X, bt, y_pad)
    return out.reshape(BATCH)


# speedup vs baseline: 3.0701x; 1.0083x over previous
import functools

import jax
import jax.numpy as jnp
from jax.experimental import pallas as pl
from jax.experimental.pallas import tpu as pltpu

RADIUS = 2.0
Y_ACC = 0.001
X_ACC = 0.001
NUM_DIMS = 8
TEMP = 50.0
EPS = 0.0001
BATCH = 2048
NY = int(2 * RADIUS / Y_ACC) + 1
NX = int(2 * RADIUS / X_ACC) + 1
NY_PAD = 4096
BLK_B = 64


def _fsm_kernel(x_ref, b_ref, y_ref, out_ref):
    x = x_ref[...]
    xp = jnp.clip(x, -RADIUS + EPS, RADIUS - EPS)
    idx = jnp.round((xp + RADIUS) / (2.0 * RADIUS) * (NX - 1))
    xg = -RADIUS + idx * (2.0 * RADIUS / (NX - 1))

    y = y_ref[...]
    bt = b_ref[...]

    tm = TEMP * (1.0 - jnp.min(bt, axis=-1, keepdims=True))

    d = xg[:, :, None] - y[None, :, :]
    s = jnp.exp(-d * d) - bt[None, :, :]
    e = jnp.exp(TEMP * s - tm[None, :, :])
    num = jnp.sum(e * s, axis=-1)
    den = jnp.sum(e, axis=-1)
    out_ref[...] = jnp.sum(num / den, axis=-1, keepdims=True)


@jax.jit
def kernel(X, theta):
    y_grid = jnp.linspace(-RADIUS, RADIUS, NY, dtype=jnp.float32)
    y_pad = jnp.zeros((1, NY_PAD), jnp.float32).at[0, :NY].set(y_grid)
    b = jnp.concatenate(
        [jnp.zeros((1, NUM_DIMS), jnp.float32), theta], axis=0
    )
    bt = jnp.full((NUM_DIMS, NY_PAD), 1e30, jnp.float32).at[:, :NY].set(b.T)

    grid = BATCH // BLK_B
    out = pl.pallas_call(
        _fsm_kernel,
        grid=(grid,),
        in_specs=[
            pl.BlockSpec((BLK_B, NUM_DIMS), lambda i: (i, 0)),
            pl.BlockSpec((NUM_DIMS, NY_PAD), lambda i: (0, 0)),
            pl.BlockSpec((1, NY_PAD), lambda i: (0, 0)),
        ],
        out_specs=pl.BlockSpec((BLK_B, 1), lambda i: (i, 0)),
        out_shape=jax.ShapeDtypeStruct((BATCH, 1), jnp.float32),
        compiler_params=pltpu.CompilerParams(
            dimension_semantics=("parallel",),
        ),
    )(X, bt, y_pad)
    return out.reshape(BATCH)


# speedup vs baseline: 3.2588x; 1.0615x over previous
import functools

import jax
import jax.numpy as jnp
from jax.experimental import pallas as pl
from jax.experimental.pallas import tpu as pltpu

RADIUS = 2.0
Y_ACC = 0.001
X_ACC = 0.001
NUM_DIMS = 8
TEMP = 50.0
EPS = 0.0001
BATCH = 2048
NY = int(2 * RADIUS / Y_ACC) + 1
NX = int(2 * RADIUS / X_ACC) + 1
NY_PAD = 4096
BLK_B = 64


def _fsm_kernel(x_ref, b_ref, y_ref, out_ref):
    x = x_ref[...]
    xp = jnp.clip(x, -RADIUS + EPS, RADIUS - EPS)
    idx = jnp.round((xp + RADIUS) / (2.0 * RADIUS) * (NX - 1))
    xg = -RADIUS + idx * (2.0 * RADIUS / (NX - 1))

    y = y_ref[...]
    bt = b_ref[...]

    l2e = 1.4426950408889634
    tl2e = TEMP * l2e
    tm = TEMP * (1.0 - jnp.min(bt, axis=-1, keepdims=True))
    c2 = l2e * (TEMP * bt + tm)

    d = xg[:, :, None] - y[None, :, :]
    dd = d * d
    q = jnp.exp2(dd * (-l2e))
    s = q - bt[None, :, :]
    e = jnp.exp2(q * tl2e - c2[None, :, :])
    num = jnp.sum(e * s, axis=-1)
    den = jnp.sum(e, axis=-1)
    out_ref[...] = jnp.sum(num / den, axis=-1, keepdims=True)


@jax.jit
def kernel(X, theta):
    y_grid = jnp.linspace(-RADIUS, RADIUS, NY, dtype=jnp.float32)
    y_pad = jnp.zeros((1, NY_PAD), jnp.float32).at[0, :NY].set(y_grid)
    b = jnp.concatenate(
        [jnp.zeros((1, NUM_DIMS), jnp.float32), theta], axis=0
    )
    bt = jnp.full((NUM_DIMS, NY_PAD), 1e30, jnp.float32).at[:, :NY].set(b.T)

    grid = BATCH // BLK_B
    out = pl.pallas_call(
        _fsm_kernel,
        grid=(grid,),
        in_specs=[
            pl.BlockSpec((BLK_B, NUM_DIMS), lambda i: (i, 0)),
            pl.BlockSpec((NUM_DIMS, NY_PAD), lambda i: (0, 0)),
            pl.BlockSpec((1, NY_PAD), lambda i: (0, 0)),
        ],
        out_specs=pl.BlockSpec((BLK_B, 1), lambda i: (i, 0)),
        out_shape=jax.ShapeDtypeStruct((BATCH, 1), jnp.float32),
        compiler_params=pltpu.CompilerParams(
            dimension_semantics=("parallel",),
        ),
    )(X, bt, y_pad)
    return out.reshape(BATCH)
